# baseline (speedup 1.0000x reference)
import jax
import jax.numpy as jnp
from jax import lax
from jax.experimental import pallas as pl

GAMMA = 0.99

B = 16384
A = 1024
H = A // 2
RB = 1024
NBLK = B // RB


def _body(acl_ref, acr_ref, avl_ref, avr_ref, rew_ref, done_ref, out_ref):
    avl = avl_ref[...]
    avr = avr_ref[...]
    mx = jnp.maximum(jnp.max(avl, axis=1, keepdims=True),
                     jnp.max(avr, axis=1, keepdims=True))
    cols = lax.broadcasted_iota(jnp.int32, (RB, H), 1)
    big = jnp.int32(2**30)
    candl = jnp.where(avl == mx, cols, big)
    candr = jnp.where(avr == mx, cols + H, big)
    cstar = jnp.minimum(jnp.min(candl, axis=1, keepdims=True),
                        jnp.min(candr, axis=1, keepdims=True))
    zero = jnp.float32(0.0)
    val = (jnp.sum(jnp.where(candl == cstar, acl_ref[...], zero),
                   axis=1, keepdims=True)
           + jnp.sum(jnp.where(candr == cstar, acr_ref[...], zero),
                     axis=1, keepdims=True))
    vl = val.reshape(1, 1, RB)
    dl = done_ref[...].reshape(1, 1, RB)
    rl = rew_ref[...].reshape(1, 1, RB)
    w = jnp.where(dl != zero, zero, vl)
    out_ref[...] = w * jnp.float32(GAMMA) + rl


def kernel(next_state_actual_values, next_state_action_values, reward, is_done):
    rew1 = reward.reshape(B)
    done1 = is_done.astype(jnp.float32).reshape(B)
    out = pl.pallas_call(
        _body,
        grid=(NBLK,),
        in_specs=[
            pl.BlockSpec((RB, H), lambda i: (i, 0)),
            pl.BlockSpec((RB, H), lambda i: (i, 1)),
            pl.BlockSpec((RB, H), lambda i: (i, 0)),
            pl.BlockSpec((RB, H), lambda i: (i, 1)),
            pl.BlockSpec((RB,), lambda i: (i,)),
            pl.BlockSpec((RB,), lambda i: (i,)),
        ],
        out_specs=pl.BlockSpec((1, 1, RB), lambda i: (i, 0, 0)),
        out_shape=jax.ShapeDtypeStruct((NBLK, 1, RB), jnp.float32),
    )(next_state_actual_values, next_state_actual_values,
      next_state_action_values, next_state_action_values, rew1, done1)
    return out.reshape(B)


# speedup vs baseline: 1.0646x; 1.0646x over previous
import jax
import jax.numpy as jnp
from jax import lax
from jax.experimental import pallas as pl

GAMMA = 0.99

B = 16384
A = 1024
RB = 1024
NBLK = B // RB


def _body(actual_ref, action_ref, rew_ref, done_ref, out_ref):
    av = action_ref[...]
    ac = actual_ref[...]
    mx = jnp.max(av, axis=1, keepdims=True)
    cols = lax.broadcasted_iota(jnp.int32, (RB, A), 1)
    big = jnp.int32(2**30)
    cand = jnp.where(av == mx, cols, big)
    cstar = jnp.min(cand, axis=1, keepdims=True)
    val = jnp.sum(jnp.where(cand == cstar, ac, jnp.float32(0.0)),
                  axis=1, keepdims=True)
    vl = val.reshape(1, 1, RB)
    dl = done_ref[...].reshape(1, 1, RB)
    rl = rew_ref[...].reshape(1, 1, RB)
    w = jnp.where(dl != jnp.float32(0.0), jnp.float32(0.0), vl)
    out_ref[...] = w * jnp.float32(GAMMA) + rl


def kernel(next_state_actual_values, next_state_action_values, reward, is_done):
    rew1 = reward.reshape(B)
    done1 = is_done.astype(jnp.float32).reshape(B)
    out = pl.pallas_call(
        _body,
        grid=(NBLK,),
        in_specs=[
            pl.BlockSpec((RB, A), lambda i: (i, 0)),
            pl.BlockSpec((RB, A), lambda i: (i, 0)),
            pl.BlockSpec((RB,), lambda i: (i,)),
            pl.BlockSpec((RB,), lambda i: (i,)),
        ],
        out_specs=pl.BlockSpec((1, 1, RB), lambda i: (i, 0, 0)),
        out_shape=jax.ShapeDtypeStruct((NBLK, 1, RB), jnp.float32),
    )(next_state_actual_values, next_state_action_values, rew1, done1)
    return out.reshape(B)


# speedup vs baseline: 1.0785x; 1.0131x over previous
import jax
import jax.numpy as jnp
from jax import lax
from jax.experimental import pallas as pl

GAMMA = 0.99

B = 16384
A = 1024
RB = 2048
NBLK = B // RB


def _body(actual_ref, action_ref, rew_ref, done_ref, out_ref):
    av = action_ref[...]
    ac = actual_ref[...]
    mx = jnp.max(av, axis=1, keepdims=True)
    cols = lax.broadcasted_iota(jnp.int32, (RB, A), 1)
    big = jnp.int32(2**30)
    cand = jnp.where(av == mx, cols, big)
    cstar = jnp.min(cand, axis=1, keepdims=True)
    val = jnp.sum(jnp.where(cand == cstar, ac, jnp.float32(0.0)),
                  axis=1, keepdims=True)
    vl = val.reshape(1, 1, RB)
    dl = done_ref[...].reshape(1, 1, RB)
    rl = rew_ref[...].reshape(1, 1, RB)
    w = jnp.where(dl != jnp.float32(0.0), jnp.float32(0.0), vl)
    out_ref[...] = w * jnp.float32(GAMMA) + rl


def kernel(next_state_actual_values, next_state_action_values, reward, is_done):
    rew1 = reward.reshape(B)
    done1 = is_done.astype(jnp.float32).reshape(B)
    out = pl.pallas_call(
        _body,
        grid=(NBLK,),
        in_specs=[
            pl.BlockSpec((RB, A), lambda i: (i, 0)),
            pl.BlockSpec((RB, A), lambda i: (i, 0)),
            pl.BlockSpec((RB,), lambda i: (i,)),
            pl.BlockSpec((RB,), lambda i: (i,)),
        ],
        out_specs=pl.BlockSpec((1, 1, RB), lambda i: (i, 0, 0)),
        out_shape=jax.ShapeDtypeStruct((NBLK, 1, RB), jnp.float32),
    )(next_state_actual_values, next_state_action_values, rew1, done1)
    return out.reshape(B)
